# baseline (speedup 1.0000x reference)
import functools
import math

import jax
import jax.numpy as jnp
import numpy as np
from jax import lax
from jax.experimental import pallas as pl
from jax.experimental.pallas import tpu as pltpu
from jax.experimental.pallas import tpu_sc as plsc

B = 1024
L = 200
D = 128
NC = 2
NS = 16
NW = NC * NS
N = B * L
ROWS_PER_W = N // NW
SEQ_PER_W = ROWS_PER_W // L
SCALE = float(np.sqrt(float(D)))
LANES = 16


def _pos_encoding(length, depth):
    half = depth / 2
    positions = np.arange(length)[:, np.newaxis]
    depths = np.arange(half)[np.newaxis, :] / half
    angle_rates = 1.0 / np.power(10000.0, depths)
    angle_rads = positions * angle_rates
    return np.concatenate(
        [np.sin(angle_rads), np.cos(angle_rads)], axis=-1
    ).astype(np.float32)


_PE = _pos_encoding(L, D)


def _sc_body(table, xflat, pe, out, idx_v, pe_v, rows_v, semg):
    wid = lax.axis_index("s") * NC + lax.axis_index("c")
    base = wid * ROWS_PER_W

    pltpu.sync_copy(xflat.at[pl.ds(base, ROWS_PER_W)], idx_v)
    pltpu.sync_copy(pe, pe_v)

    @pl.loop(0, SEQ_PER_W)
    def _seq(s):
        off = s * L
        cp1 = pltpu.async_copy(
            table.at[idx_v.at[pl.ds(off, 128)]],
            rows_v.at[pl.ds(0, 128)],
            semg,
        )
        cp2 = pltpu.async_copy(
            table.at[idx_v.at[pl.ds(off + 128, L - 128)]],
            rows_v.at[pl.ds(128, L - 128)],
            semg,
        )
        cp1.wait()
        cp2.wait()

        @pl.loop(0, L)
        def _row(i):
            for d in range(D // LANES):
                sl = pl.ds(d * LANES, LANES)
                rows_v[i, sl] = rows_v[i, sl] * SCALE + pe_v[i, sl]

        pltpu.sync_copy(rows_v, out.at[pl.ds(base + off, L)])


@functools.partial(jax.jit, static_argnames=())
def kernel(x, table):
    xflat = x.reshape(N)
    pe = jnp.asarray(_PE)
    mesh = plsc.VectorSubcoreMesh(core_axis_name="c", subcore_axis_name="s")
    out = pl.kernel(
        _sc_body,
        out_type=jax.ShapeDtypeStruct((N, D), jnp.float32),
        mesh=mesh,
        scratch_types=[
            pltpu.VMEM((ROWS_PER_W,), jnp.int32),
            pltpu.VMEM((L, D), jnp.float32),
            pltpu.VMEM((L, D), jnp.float32),
            pltpu.SemaphoreType.DMA,
        ],
    )(table, xflat, pe)
    return out.reshape(B, L, D)
